# baseline (speedup 1.0000x reference)
import jax
import jax.numpy as jnp
from jax import lax
from jax.experimental import pallas as pl
from jax.experimental.pallas import tpu as pltpu
from jax.experimental.pallas import tpu_sc as plsc

NC = 2
NS = 16
NW = NC * NS

B = 128
ROWS_PER_B = 384
CHUNK = 48
NCHUNK = ROWS_PER_B // CHUNK
B_PER_W = B // NW
TABLE_ROWS = 6 * ROWS_PER_B
ROW_W = 1024


def _sc_body(table_hbm, idx_hbm, out_hbm, idx_v, buf0, buf1, sem0, sem1):
    cid = lax.axis_index("c")
    sid = lax.axis_index("s")
    wid = sid * NC + cid

    pltpu.sync_copy(idx_hbm.at[pl.ds(wid * B_PER_W, B_PER_W)], idx_v)

    bufs = (buf0, buf1)
    sems = (sem0, sem1)

    for bi in range(B_PER_W):
        b = wid * B_PER_W + bi
        descs = {}
        descs[0] = pltpu.async_copy(
            table_hbm.at[idx_v.at[bi, 0]], bufs[0], sems[0])
        for g in range(NCHUNK):
            if g + 1 < NCHUNK:
                descs[g + 1] = pltpu.async_copy(
                    table_hbm.at[idx_v.at[bi, g + 1]],
                    bufs[(g + 1) % 2], sems[(g + 1) % 2])
            descs[g].wait()
            pltpu.sync_copy(
                bufs[g % 2],
                out_hbm.at[pl.ds(b * ROWS_PER_B + g * CHUNK, CHUNK)])


def kernel(batch_size, cat_id, embeddings):
    table2d = embeddings.reshape(TABLE_ROWS, ROW_W)
    r = jnp.arange(ROWS_PER_B, dtype=jnp.int32)
    perm = (r % 3) * 128 + r // 3
    idx_all = (cat_id.astype(jnp.int32) * ROWS_PER_B)[:, None] + perm[None, :]
    idx_all = idx_all.reshape(B, NCHUNK, CHUNK)

    mesh = plsc.VectorSubcoreMesh(core_axis_name="c", subcore_axis_name="s")
    out2d = pl.kernel(
        _sc_body,
        out_type=jax.ShapeDtypeStruct((B * ROWS_PER_B, ROW_W), jnp.float32),
        mesh=mesh,
        scratch_types=[
            pltpu.VMEM((B_PER_W, NCHUNK, CHUNK), jnp.int32),
            pltpu.VMEM((CHUNK, ROW_W), jnp.float32),
            pltpu.VMEM((CHUNK, ROW_W), jnp.float32),
            pltpu.SemaphoreType.DMA,
            pltpu.SemaphoreType.DMA,
        ],
    )(table2d, idx_all)
    return out2d.reshape(B, 128, 3072)
